# baseline (speedup 1.0000x reference)
import functools
import math

import jax
import jax.numpy as jnp
import numpy as np
from jax import lax
from jax.experimental import pallas as pl
from jax.experimental.pallas import tpu as pltpu
from jax.experimental.pallas import tpu_sc as plsc

_D_MODEL = 1024
_SEQ_LEN = 2048
_BATCH = 4
_MAX_LEN = 5000

_NC = 2
_NS = 16
_NW = _NC * _NS

_POS_PER_W = _SEQ_LEN // _NW
_CHUNK = 16
_NCHUNK = _POS_PER_W // _CHUNK
_VECS_PER_ROW = _D_MODEL // 16


def _make_pe(d_model, seq_len):
    position = np.arange(_MAX_LEN)[:, np.newaxis]
    div_term = np.exp(np.arange(0, d_model, 2) * (-math.log(10000.0) / d_model))
    pe = np.zeros((_MAX_LEN, d_model))
    pe[:, 0::2] = np.sin(position * div_term)
    pe[:, 1::2] = np.cos(position * div_term)
    return pe[:seq_len].astype(np.float32)


_PE = _make_pe(_D_MODEL, _SEQ_LEN)


@functools.partial(
    pl.kernel,
    mesh=plsc.VectorSubcoreMesh(core_axis_name="c", subcore_axis_name="s"),
    out_type=jax.ShapeDtypeStruct((_BATCH * _SEQ_LEN, _D_MODEL), jnp.float32),
    scratch_types=[
        pltpu.VMEM((_CHUNK,), jnp.int32),
        pltpu.VMEM((_CHUNK, _D_MODEL), jnp.float32),
        pltpu.VMEM((_CHUNK, _D_MODEL), jnp.float32),
        pltpu.SemaphoreType.DMA,
    ],
)
def _sc_embed(idx_hbm, table_hbm, pe_hbm, out_hbm, idx_v, pe_v, rows_v, sem):
    wid = lax.axis_index("s") * _NC + lax.axis_index("c")
    pos0 = wid * _POS_PER_W

    for c in range(_NCHUNK):
        pos = pos0 + c * _CHUNK
        pltpu.sync_copy(pe_hbm.at[pl.ds(pos, _CHUNK)], pe_v)
        for b in range(_BATCH):
            flat = b * _SEQ_LEN + pos
            pltpu.sync_copy(idx_hbm.at[pl.ds(flat, _CHUNK)], idx_v)
            pltpu.async_copy(table_hbm.at[idx_v], rows_v, sem).wait()

            def _add_row(r, _):
                for k in range(_VECS_PER_ROW):
                    sl = pl.ds(k * 16, 16)
                    rows_v[r, sl] = rows_v[r, sl] + pe_v[r, sl]
                return 0

            lax.fori_loop(0, _CHUNK, _add_row, 0)
            pltpu.sync_copy(rows_v, out_hbm.at[pl.ds(flat, _CHUNK)])


def kernel(x, emb_table):
    batch, seq_len = x.shape
    idx = x.reshape(-1).astype(jnp.int32)
    out = _sc_embed(idx, emb_table, jnp.asarray(_PE))
    return out.reshape(batch, seq_len, emb_table.shape[1])


# speedup vs baseline: 1.4317x; 1.4317x over previous
import functools
import math

import jax
import jax.numpy as jnp
import numpy as np
from jax import lax
from jax.experimental import pallas as pl
from jax.experimental.pallas import tpu as pltpu
from jax.experimental.pallas import tpu_sc as plsc

_D_MODEL = 1024
_SEQ_LEN = 2048
_BATCH = 4
_MAX_LEN = 5000

_NC = 2
_NS = 16
_NW = _NC * _NS

_POS_PER_W = _SEQ_LEN // _NW
_CHUNK = 16
_NCHUNK = _POS_PER_W // _CHUNK
_NSTEP = _NCHUNK * _BATCH
_VECS_PER_ROW = _D_MODEL // 16


def _make_pe(d_model, seq_len):
    position = np.arange(_MAX_LEN)[:, np.newaxis]
    div_term = np.exp(np.arange(0, d_model, 2) * (-math.log(10000.0) / d_model))
    pe = np.zeros((_MAX_LEN, d_model))
    pe[:, 0::2] = np.sin(position * div_term)
    pe[:, 1::2] = np.cos(position * div_term)
    return pe[:seq_len].astype(np.float32)


_PE = _make_pe(_D_MODEL, _SEQ_LEN)


@functools.partial(
    pl.kernel,
    mesh=plsc.VectorSubcoreMesh(core_axis_name="c", subcore_axis_name="s"),
    out_type=jax.ShapeDtypeStruct((_BATCH * _SEQ_LEN, _D_MODEL), jnp.float32),
    scratch_types=[
        pltpu.VMEM((_NSTEP, _CHUNK), jnp.int32),
        pltpu.VMEM((_CHUNK, _D_MODEL), jnp.float32),
        pltpu.VMEM((_CHUNK, _D_MODEL), jnp.float32),
        pltpu.VMEM((_CHUNK, _D_MODEL), jnp.float32),
        pltpu.VMEM((_CHUNK, _D_MODEL), jnp.float32),
        pltpu.SemaphoreType.DMA,
        pltpu.SemaphoreType.DMA,
        pltpu.SemaphoreType.DMA,
        pltpu.SemaphoreType.DMA,
        pltpu.SemaphoreType.DMA,
        pltpu.SemaphoreType.DMA,
    ],
)
def _sc_embed(idx_hbm, table_hbm, pe_hbm, out_hbm,
              idx_v, r0, r1, p0, p1,
              gs0, gs1, os0, os1, ps0, ps1):
    wid = lax.axis_index("s") * _NC + lax.axis_index("c")
    pos0 = wid * _POS_PER_W

    rbuf = (r0, r1)
    pbuf = (p0, p1)
    gsem = (gs0, gs1)
    osem = (os0, os1)
    psem = (ps0, ps1)

    pltpu.sync_copy(idx_hbm.at[wid], idx_v)

    pe_cp = [None, None]
    pe_cp[0] = pltpu.async_copy(pe_hbm.at[pl.ds(pos0, _CHUNK)], p0, ps0)
    g_cp = [None] * _NSTEP
    o_cp = [None] * _NSTEP
    g_cp[0] = pltpu.async_copy(table_hbm.at[idx_v.at[0]], r0, gs0)

    for s in range(_NSTEP):
        c, b = divmod(s, _BATCH)
        if s + 1 < _NSTEP:
            c1, b1 = divmod(s + 1, _BATCH)
            nb = (s + 1) % 2
            if o_cp[s - 1] is not None:
                o_cp[s - 1].wait()
            if b1 == 0:
                pe_cp[c1 % 2] = pltpu.async_copy(
                    pe_hbm.at[pl.ds(pos0 + c1 * _CHUNK, _CHUNK)],
                    pbuf[c1 % 2], psem[c1 % 2])
            g_cp[s + 1] = pltpu.async_copy(
                table_hbm.at[idx_v.at[s + 1]], rbuf[nb], gsem[nb])
        if b == 0:
            pe_cp[c % 2].wait()
        g_cp[s].wait()

        rb = rbuf[s % 2]
        pb = pbuf[c % 2]

        def _add_row(r, _):
            for k in range(_VECS_PER_ROW):
                sl = pl.ds(k * 16, 16)
                rb[r, sl] = rb[r, sl] + pb[r, sl]
            return 0

        lax.fori_loop(0, _CHUNK, _add_row, 0)
        o_cp[s] = pltpu.async_copy(
            rb, out_hbm.at[pl.ds(b * _SEQ_LEN + pos0 + c * _CHUNK, _CHUNK)],
            osem[s % 2])

    o_cp[_NSTEP - 2].wait()
    o_cp[_NSTEP - 1].wait()


def kernel(x, emb_table):
    batch, seq_len = x.shape
    d_model = emb_table.shape[1]
    idx = (x.astype(jnp.int32)
           .reshape(batch, _NW, _NCHUNK, _CHUNK)
           .transpose(1, 2, 0, 3)
           .reshape(_NW, _NSTEP, _CHUNK))
    out = _sc_embed(idx, emb_table, jnp.asarray(_PE))
    return out.reshape(batch, seq_len, d_model)


# speedup vs baseline: 1.6149x; 1.1280x over previous
import functools
import math

import jax
import jax.numpy as jnp
import numpy as np
from jax import lax
from jax.experimental import pallas as pl
from jax.experimental.pallas import tpu as pltpu
from jax.experimental.pallas import tpu_sc as plsc

_D_MODEL = 1024
_SEQ_LEN = 2048
_BATCH = 4
_MAX_LEN = 5000

_NC = 2
_NS = 16
_NW = _NC * _NS

_POS_PER_W = _SEQ_LEN // _NW
_CHUNK = 16
_NCHUNK = _POS_PER_W // _CHUNK
_NSTEP = _NCHUNK * _BATCH
_VECS_PER_ROW = _D_MODEL // 16


def _make_pe(d_model, seq_len):
    position = np.arange(_MAX_LEN)[:, np.newaxis]
    div_term = np.exp(np.arange(0, d_model, 2) * (-math.log(10000.0) / d_model))
    pe = np.zeros((_MAX_LEN, d_model))
    pe[:, 0::2] = np.sin(position * div_term)
    pe[:, 1::2] = np.cos(position * div_term)
    return pe[:seq_len].astype(np.float32)


_PE = _make_pe(_D_MODEL, _SEQ_LEN)


@functools.partial(
    pl.kernel,
    mesh=plsc.VectorSubcoreMesh(core_axis_name="c", subcore_axis_name="s"),
    out_type=jax.ShapeDtypeStruct((_BATCH * _SEQ_LEN, _D_MODEL), jnp.float32),
    scratch_types=[
        pltpu.VMEM((_NSTEP, _CHUNK), jnp.int32),
        pltpu.VMEM((_CHUNK, _D_MODEL), jnp.float32),
        pltpu.VMEM((_CHUNK, _D_MODEL), jnp.float32),
        pltpu.VMEM((_CHUNK, _D_MODEL), jnp.float32),
        pltpu.VMEM((_CHUNK, _D_MODEL), jnp.float32),
        pltpu.VMEM((_CHUNK, _D_MODEL), jnp.float32),
        pltpu.VMEM((_CHUNK, _D_MODEL), jnp.float32),
        pltpu.SemaphoreType.DMA,
        pltpu.SemaphoreType.DMA,
        pltpu.SemaphoreType.DMA,
        pltpu.SemaphoreType.DMA,
        pltpu.SemaphoreType.DMA,
        pltpu.SemaphoreType.DMA,
        pltpu.SemaphoreType.DMA,
        pltpu.SemaphoreType.DMA,
        pltpu.SemaphoreType.DMA,
        pltpu.SemaphoreType.DMA,
    ],
)
def _sc_embed(idx_hbm, table_hbm, pe_hbm, out_hbm,
              idx_v, r0, r1, r2, r3, p0, p1,
              gs0, gs1, gs2, gs3, os0, os1, os2, os3, ps0, ps1):
    wid = lax.axis_index("s") * _NC + lax.axis_index("c")
    pos0 = wid * _POS_PER_W

    rbuf = (r0, r1, r2, r3)
    pbuf = (p0, p1)
    gsem = (gs0, gs1, gs2, gs3)
    osem = (os0, os1, os2, os3)
    psem = (ps0, ps1)
    nbuf = 4

    pltpu.sync_copy(idx_hbm.at[wid], idx_v)

    pe_cp = [None, None]
    pe_cp[0] = pltpu.async_copy(pe_hbm.at[pl.ds(pos0, _CHUNK)], p0, ps0)
    g_cp = [None] * _NSTEP
    o_cp = [None] * _NSTEP
    for t in range(nbuf - 1):
        g_cp[t] = pltpu.async_copy(
            table_hbm.at[idx_v.at[t]], rbuf[t], gsem[t])

    for s in range(_NSTEP):
        c, b = divmod(s, _BATCH)
        g_cp[s].wait()
        if b == 0:
            pe_cp[c % 2].wait()

        rb = rbuf[s % nbuf]
        pb = pbuf[c % 2]

        def _add_row(r, _):
            for k in range(_VECS_PER_ROW):
                sl = pl.ds(k * 16, 16)
                rb[r, sl] = rb[r, sl] + pb[r, sl]
            return 0

        lax.fori_loop(0, _CHUNK, _add_row, 0)
        o_cp[s] = pltpu.async_copy(
            rb, out_hbm.at[pl.ds(b * _SEQ_LEN + pos0 + c * _CHUNK, _CHUNK)],
            osem[s % nbuf])

        t = s + nbuf - 1
        if t < _NSTEP:
            if s >= 1:
                o_cp[s - 1].wait()
            c1, b1 = divmod(t, _BATCH)
            if b1 == 0:
                pe_cp[c1 % 2] = pltpu.async_copy(
                    pe_hbm.at[pl.ds(pos0 + c1 * _CHUNK, _CHUNK)],
                    pbuf[c1 % 2], psem[c1 % 2])
            g_cp[t] = pltpu.async_copy(
                table_hbm.at[idx_v.at[t]], rbuf[t % nbuf], gsem[t % nbuf])

    for s in range(_NSTEP - nbuf, _NSTEP):
        o_cp[s].wait()


def kernel(x, emb_table):
    batch, seq_len = x.shape
    d_model = emb_table.shape[1]
    idx = (x.astype(jnp.int32)
           .reshape(batch, _NW, _NCHUNK, _CHUNK)
           .transpose(1, 2, 0, 3)
           .reshape(_NW, _NSTEP, _CHUNK))
    out = _sc_embed(idx, emb_table, jnp.asarray(_PE))
    return out.reshape(batch, seq_len, d_model)
